# baseline (speedup 1.0000x reference)
import functools

import jax
import jax.numpy as jnp
from jax import lax
from jax.experimental import pallas as pl
from jax.experimental.pallas import tpu as pltpu
from jax.experimental.pallas import tpu_sc as plsc

_EPS = 1e-7
_NW = 32


@functools.cache
def _make_sc_gather(seg, hid, chunk):
    per_w = seg // _NW
    nchunks = per_w // chunk
    mesh = plsc.VectorSubcoreMesh(core_axis_name="c", subcore_axis_name="s")

    @functools.partial(
        pl.kernel,
        out_type=jax.ShapeDtypeStruct((seg, hid), jnp.float32),
        mesh=mesh,
        compiler_params=pltpu.CompilerParams(needs_layout_passes=False),
        scratch_types=[
            pltpu.VMEM((chunk,), jnp.int32),
            pltpu.VMEM((chunk, hid), jnp.float32),
            pltpu.VMEM((chunk,), jnp.int32),
            pltpu.VMEM((chunk, hid), jnp.float32),
            pltpu.SemaphoreType.DMA,
            pltpu.SemaphoreType.DMA,
        ],
    )
    def body(ids_hbm, word_hbm, out_hbm, idx0, rows0, idx1, rows1, sem0, sem1):
        wid = lax.axis_index("s") * 2 + lax.axis_index("c")
        base = wid * per_w
        idx = (idx0, idx1)
        rows = (rows0, rows1)
        sems = (sem0, sem1)

        def start(g):
            c0 = pl.multiple_of(base + g * chunk, chunk)
            b = g % 2
            pltpu.sync_copy(ids_hbm.at[pl.ds(c0, chunk)], idx[b])
            return pltpu.async_copy(word_hbm.at[idx[b]], rows[b], sems[b])

        cp = start(0)
        for g in range(nchunks):
            nxt = start(g + 1) if g + 1 < nchunks else None
            cp.wait()
            c0 = pl.multiple_of(base + g * chunk, chunk)
            pltpu.sync_copy(rows[g % 2], out_hbm.at[pl.ds(c0, chunk)])
            cp = nxt

    return body


@functools.cache
def _make_tc_ln(seg, hid, blk):
    def body(g_ref, pos_ref, gam_ref, bet_ref, mask_ref, o_ref):
        x = g_ref[...] + pos_ref[...]
        mu = jnp.mean(x, axis=-1, keepdims=True)
        var = jnp.mean((x - mu) * (x - mu), axis=-1, keepdims=True)
        y = (x - mu) * lax.rsqrt(var + _EPS)
        y = y * gam_ref[...] + bet_ref[...]
        o_ref[...] = y * mask_ref[...]

    grid = (seg // blk,)
    return pl.pallas_call(
        body,
        grid=grid,
        in_specs=[
            pl.BlockSpec((blk, hid), lambda i: (i, 0)),
            pl.BlockSpec((blk, hid), lambda i: (i, 0)),
            pl.BlockSpec((1, hid), lambda i: (0, 0)),
            pl.BlockSpec((1, hid), lambda i: (0, 0)),
            pl.BlockSpec((blk, 1), lambda i: (i, 0)),
        ],
        out_specs=pl.BlockSpec((blk, hid), lambda i: (i, 0)),
        out_shape=jax.ShapeDtypeStruct((seg, hid), jnp.float32),
    )


def kernel(input_ids, mask, word_table, pos_table, gamma, beta):
    b, s = input_ids.shape
    _, hid = word_table.shape
    ids = input_ids.astype(jnp.int32)
    mk = mask.astype(jnp.float32)
    gam2 = gamma.reshape(1, hid)
    bet2 = beta.reshape(1, hid)

    gather = _make_sc_gather(s, hid, 32)
    ln = _make_tc_ln(s, hid, 256)

    outs = []
    for k in range(b):
        rows = gather(ids[k], word_table)
        outs.append(ln(rows, pos_table, gam2, bet2, mk[k].reshape(s, 1)))
    return jnp.stack(outs).reshape(b, s, hid)


# speedup vs baseline: 1.4264x; 1.4264x over previous
import functools

import jax
import jax.numpy as jnp
from jax import lax
from jax.experimental import pallas as pl
from jax.experimental.pallas import tpu as pltpu
from jax.experimental.pallas import tpu_sc as plsc

_EPS = 1e-7
_NW = 32


@functools.cache
def _make_sc_gather(seg, hid, chunk):
    per_w = seg // _NW
    nchunks = per_w // chunk
    mesh = plsc.VectorSubcoreMesh(core_axis_name="c", subcore_axis_name="s")

    @functools.partial(
        pl.kernel,
        out_type=jax.ShapeDtypeStruct((seg, hid), jnp.float32),
        mesh=mesh,
        compiler_params=pltpu.CompilerParams(needs_layout_passes=False),
        scratch_types=[
            pltpu.VMEM((chunk,), jnp.int32),
            pltpu.VMEM((chunk, hid), jnp.float32),
            pltpu.VMEM((chunk,), jnp.int32),
            pltpu.VMEM((chunk, hid), jnp.float32),
            pltpu.SemaphoreType.DMA,
            pltpu.SemaphoreType.DMA,
        ],
    )
    def body(ids_hbm, word_hbm, out_hbm, idx0, rows0, idx1, rows1, sem0, sem1):
        wid = lax.axis_index("s") * 2 + lax.axis_index("c")
        base = wid * per_w
        idx = (idx0, idx1)
        rows = (rows0, rows1)
        sems = (sem0, sem1)

        def start(g):
            c0 = pl.multiple_of(base + g * chunk, chunk)
            b = g % 2
            pltpu.sync_copy(ids_hbm.at[pl.ds(c0, chunk)], idx[b])
            return pltpu.async_copy(word_hbm.at[idx[b]], rows[b], sems[b])

        cp = start(0)
        for g in range(nchunks):
            nxt = start(g + 1) if g + 1 < nchunks else None
            cp.wait()
            c0 = pl.multiple_of(base + g * chunk, chunk)
            pltpu.sync_copy(rows[g % 2], out_hbm.at[pl.ds(c0, chunk)])
            cp = nxt

    return body


@functools.cache
def _make_tc_ln(ntok, b, w, hid, j, nseg, first):
    sblocks = nseg

    def body(g_ref, pos_ref, gam_ref, bet_ref, mask_ref, *rest):
        o_ref = rest[-1]
        x = g_ref[...] + pos_ref[...]
        mu = jnp.mean(x, axis=-1, keepdims=True)
        var = jnp.mean((x - mu) * (x - mu), axis=-1, keepdims=True)
        y = (x - mu) * lax.rsqrt(var + _EPS)
        y = y * gam_ref[...] + bet_ref[...]
        o_ref[...] = y * mask_ref[...]

    in_specs = [
        pl.BlockSpec((w, hid), lambda i: (i, 0)),
        pl.BlockSpec((w, hid), lambda i, j=j: (j, 0)),
        pl.BlockSpec((1, hid), lambda i: (0, 0)),
        pl.BlockSpec((1, hid), lambda i: (0, 0)),
        pl.BlockSpec((w, 1), lambda i, j=j: (i * sblocks + j, 0)),
    ]
    kwargs = {}
    if not first:
        in_specs.append(pl.BlockSpec(memory_space=pltpu.MemorySpace.HBM))
        kwargs["input_output_aliases"] = {5: 0}
    return pl.pallas_call(
        body,
        grid=(b,),
        in_specs=in_specs,
        out_specs=pl.BlockSpec((w, hid), lambda i, j=j: (i * sblocks + j, 0)),
        out_shape=jax.ShapeDtypeStruct((ntok, hid), jnp.float32),
        **kwargs,
    )


def kernel(input_ids, mask, word_table, pos_table, gamma, beta):
    b, s = input_ids.shape
    _, hid = word_table.shape
    nseg = 4
    w = s // nseg
    seg = b * w
    ids = input_ids.astype(jnp.int32)
    ids_sm = ids.reshape(b, nseg, w).transpose(1, 0, 2).reshape(nseg, seg)
    mk = mask.astype(jnp.float32).reshape(b * s, 1)
    gam2 = gamma.reshape(1, hid)
    bet2 = beta.reshape(1, hid)

    gather = _make_sc_gather(seg, hid, 32)
    rows = [gather(ids_sm[j], word_table) for j in range(nseg)]

    out = None
    for j in range(nseg):
        ln = _make_tc_ln(b * s, b, w, hid, j, nseg, first=(j == 0))
        args = (rows[j], pos_table, gam2, bet2, mk)
        out = ln(*args) if j == 0 else ln(*args, out)
    return out.reshape(b, s, hid)
